# baseline (speedup 1.0000x reference)
import jax
import jax.numpy as jnp
from jax.experimental import pallas as pl


def _bn(x, g, b):
    m = x.mean(0)
    v = x.var(0)
    return g * (x - m) / jnp.sqrt(v + 1e-5) + b


def _copy_kernel(x_ref, o_ref):
    o_ref[...] = x_ref[...]


def kernel(vocab_datas, x, edge_index, edge_attr, inter_tensor, smiles_id, batch,
           emb_table, node_W, node_b, node_gamma, node_beta, atom_W, atom_b,
           gnn_W1, gnn_b1, gnn_W2, gnn_b2, gnn_gamma, gnn_beta, pred_W, pred_b):
    N = x.shape[0]
    G = smiles_id.shape[0]
    L = gnn_W1.shape[0]
    xs = jnp.squeeze(x)
    x_embed = jnp.take(emb_table, xs, axis=0)
    x_input = jnp.take(vocab_datas, xs, axis=0)
    h = jnp.concatenate([x_embed, x_input], axis=1) @ node_W + node_b
    h = _bn(jax.nn.relu(h), node_gamma, node_beta)
    e0 = jnp.concatenate([inter_tensor[:, 0, :], edge_attr[:, 0:2]], axis=1) @ atom_W + atom_b
    e1 = jnp.concatenate([inter_tensor[:, 1, :], edge_attr[:, 2:4]], axis=1) @ atom_W + atom_b
    ea = jnp.concatenate([jnp.concatenate([e0, e1], axis=1),
                          jnp.concatenate([e1, e0], axis=1)], axis=0)
    ei = jnp.concatenate([edge_index, jnp.stack([edge_index[1], edge_index[0]], axis=0)], axis=1)
    src = ei[0]
    dst = ei[1]
    for i in range(L):
        msg = jax.nn.relu(jnp.take(h, src, axis=0) + ea)
        agg = jax.ops.segment_sum(msg, dst, num_segments=N)
        z = h + agg
        z = jax.nn.relu(z @ gnn_W1[i] + gnn_b1[i]) @ gnn_W2[i] + gnn_b2[i]
        z = _bn(z, gnn_gamma[i], gnn_beta[i])
        if i < L - 1:
            z = jax.nn.relu(z)
        h = z
    node_representation = h @ pred_W + pred_b
    node_representation = pl.pallas_call(
        _copy_kernel,
        out_shape=jax.ShapeDtypeStruct(node_representation.shape, node_representation.dtype),
    )(node_representation)
    counts = jax.ops.segment_sum(jnp.ones((N,), jnp.float32), batch, num_segments=G)
    mol_pred = jax.ops.segment_sum(node_representation, batch, num_segments=G) / jnp.maximum(counts, 1.0)[:, None]
    return (mol_pred, node_representation)


# speedup vs baseline: 2.8403x; 2.8403x over previous
import functools

import jax
import jax.numpy as jnp
from jax import lax
from jax.experimental import pallas as pl
from jax.experimental.pallas import tpu as pltpu
from jax.experimental.pallas import tpu_sc as plsc

N = 10000
E = 320000
V = 5000
DV = 64
D = 128
L = 3
G = 256
DI = 16

NC = 2
NS = 16
NW = NC * NS
CE2 = 256
KCH = CE2 // 128
EPAD = 512
LROWS = 328
DUMMY = 320

N_PAD = 10240
GB = 320
GCH = 80


def _table_mm_body(x_ref, w_ref, b_ref, o_ref):
    o_ref[...] = jnp.dot(x_ref[...], w_ref[...],
                         preferred_element_type=jnp.float32) + b_ref[...]


def _fused_table(tbl_in, node_W, node_b):
    blk = 1000
    return pl.pallas_call(
        _table_mm_body,
        grid=(V // blk,),
        in_specs=[
            pl.BlockSpec((blk, DV + D), lambda i: (i, 0)),
            pl.BlockSpec((DV + D, D), lambda i: (0, 0)),
            pl.BlockSpec((1, D), lambda i: (0, 0)),
        ],
        out_specs=pl.BlockSpec((blk, D), lambda i: (i, 0)),
        out_shape=jax.ShapeDtypeStruct((V, D), jnp.float32),
    )(tbl_in, node_W, node_b.reshape(1, D))


def _edge_enc_body(x_ref, w_ref, b_ref, o_ref):
    v = jnp.dot(x_ref[...], w_ref[...],
                preferred_element_type=jnp.float32) + b_ref[...]
    o_ref[0] = v
    o_ref[1] = jnp.concatenate([v[:, 64:], v[:, :64]], axis=1)


def _edge_enc(x36, w_cat, b_cat):
    blk = 8000
    return pl.pallas_call(
        _edge_enc_body,
        grid=(E // blk,),
        in_specs=[
            pl.BlockSpec((blk, 36), lambda i: (i, 0)),
            pl.BlockSpec((36, D), lambda i: (0, 0)),
            pl.BlockSpec((1, D), lambda i: (0, 0)),
        ],
        out_specs=pl.BlockSpec((2, blk, D), lambda i: (0, i, 0)),
        out_shape=jax.ShapeDtypeStruct((2, E, D), jnp.float32),
    )(x36, w_cat, b_cat.reshape(1, D))


def _bn(x, g, b):
    m = x.mean(0)
    v = x.var(0)
    return g * (x - m) / jnp.sqrt(v + 1e-5) + b


def _mlp_body(h_ref, agg_ref, w1_ref, b1_ref, w2_ref, b2_ref, o_ref):
    z = h_ref[...] + agg_ref[...]
    t = jnp.maximum(jnp.dot(z, w1_ref[...],
                            preferred_element_type=jnp.float32) + b1_ref[...], 0.0)
    o_ref[...] = jnp.dot(t, w2_ref[...],
                         preferred_element_type=jnp.float32) + b2_ref[...]


def _mlp(h, agg, w1, b1, w2, b2):
    return pl.pallas_call(
        _mlp_body,
        out_shape=jax.ShapeDtypeStruct((N, D), jnp.float32),
    )(h, agg, w1, b1.reshape(1, 2 * D), w2, b2.reshape(1, D))


def _final_body(h_ref, pw_ref, pb_ref, bat_ref, nr_ref, mol_ref, cnt_ref):
    i = pl.program_id(0)
    nsteps = pl.num_programs(0)
    nr = jnp.dot(h_ref[...], pw_ref[...],
                 preferred_element_type=jnp.float32) + pb_ref[...]
    nr_ref[...] = nr
    bids = bat_ref[0]
    gi = lax.broadcasted_iota(jnp.int32, (G, bids.shape[1]), 0)
    oh = (bids == gi).astype(jnp.float32)

    @pl.when(i == 0)
    def _():
        mol_ref[...] = jnp.zeros_like(mol_ref)
        cnt_ref[...] = jnp.zeros_like(cnt_ref)

    mol_ref[...] += jnp.dot(oh, nr, preferred_element_type=jnp.float32,
                            precision=lax.Precision.HIGHEST)
    cnt_ref[...] += jnp.sum(oh, axis=1, keepdims=True)

    @pl.when(i == nsteps - 1)
    def _():
        mol_ref[...] = mol_ref[...] / jnp.maximum(cnt_ref[...], 1.0)


def _final(h, pred_W, pred_b, batch):
    blk = 2000
    nblk = N // blk
    bat3 = batch.reshape(nblk, 1, blk)
    return pl.pallas_call(
        _final_body,
        grid=(nblk,),
        in_specs=[
            pl.BlockSpec((blk, D), lambda i: (i, 0)),
            pl.BlockSpec((D, D), lambda i: (0, 0)),
            pl.BlockSpec((1, D), lambda i: (0, 0)),
            pl.BlockSpec((1, 1, blk), lambda i: (i, 0, 0)),
        ],
        out_specs=[
            pl.BlockSpec((blk, D), lambda i: (i, 0)),
            pl.BlockSpec((G, D), lambda i: (0, 0)),
        ],
        out_shape=[
            jax.ShapeDtypeStruct((N, D), jnp.float32),
            jax.ShapeDtypeStruct((G, D), jnp.float32),
        ],
        scratch_shapes=[pltpu.VMEM((G, 1), jnp.float32)],
    )(h, pred_W, pred_b.reshape(1, D), bat3)


_MESH = plsc.VectorSubcoreMesh(core_axis_name="c", subcore_axis_name="s")


@functools.partial(
    pl.kernel,
    out_type=jax.ShapeDtypeStruct((N_PAD, D), jnp.float32),
    mesh=_MESH,
    scratch_types=[
        pltpu.VMEM((GCH,), jnp.int32),
        pltpu.VMEM((GCH, D), jnp.float32),
        pltpu.SemaphoreType.DMA,
    ],
)
def _sc_gather(tbl_hbm, idx_hbm, out_hbm, idx_v, rows_v, sem):
    wid = lax.axis_index("s") * NC + lax.axis_index("c")
    base = wid * GB

    def chunk(j, carry):
        off = base + j * GCH
        pltpu.sync_copy(idx_hbm.at[pl.ds(off, GCH)], idx_v)
        pltpu.async_copy(tbl_hbm.at[idx_v], rows_v, sem).wait()
        pltpu.sync_copy(rows_v, out_hbm.at[pl.ds(off, GCH)])
        return carry

    lax.fori_loop(0, GB // GCH, chunk, 0)


@functools.partial(
    pl.kernel,
    out_type=jax.ShapeDtypeStruct((N, D), jnp.float32),
    mesh=_MESH,
    scratch_types=[
        pltpu.VMEM((CE2,), jnp.int32),
        pltpu.VMEM((CE2,), jnp.int32),
        pltpu.VMEM((CE2,), jnp.int32),
        pltpu.VMEM((CE2, D), jnp.float32),
        pltpu.VMEM((LROWS, D), jnp.float32),
        pltpu.VMEM((16,), jnp.int32),
        pltpu.SemaphoreType.DMA,
        pltpu.SemaphoreType.DMA,
    ],
)
def _sc_mp(h_hbm, ea2_hbm, perm_hbm, srcp_hbm, dstp_hbm, ebound_hbm, out_hbm,
           pbuf, sbuf, dbuf, ebuf, local, ebv, sema, semb):
    cid = lax.axis_index("c")
    sid = lax.axis_index("s")
    wid = sid * NC + cid

    n0 = pl.multiple_of(8 * ((wid * (N // 8)) // NW), 8)
    n1 = 8 * (((wid + 1) * (N // 8)) // NW)

    def zrow(i, c):
        for g in range(D // 16):
            local[i, pl.ds(g * 16, 16)] = jnp.zeros((16,), jnp.float32)
        return c

    lax.fori_loop(0, LROWS, zrow, 0)

    pltpu.sync_copy(ebound_hbm.at[pl.ds(pl.multiple_of(8 * wid, 8), 8)],
                    ebv.at[pl.ds(0, 8)])
    bv = ebv[pl.ds(0, 16)]
    e0 = pl.multiple_of(bv[0], 8)
    e1 = bv[1]
    nch = (e1 - e0 + CE2 - 1) // CE2

    def grp(g, c):
        dvec = dbuf[pl.ds(g * 16, 16)] - n0
        jvec = jnp.where(jnp.logical_and(dvec >= 0, dvec < DUMMY), dvec, DUMMY)
        for lane in range(16):
            j = jvec[lane]
            r = g * 16 + lane
            for q in range(D // 16):
                sl = pl.ds(q * 16, 16)
                local[j, sl] = local[j, sl] + jnp.maximum(ebuf[r, sl], 0.0)
        return c

    def chunk(cidx, carry):
        off = pl.multiple_of(e0 + cidx * CE2, 8)
        pltpu.sync_copy(perm_hbm.at[pl.ds(off, CE2)], pbuf)
        pltpu.sync_copy(srcp_hbm.at[pl.ds(off, CE2)], sbuf)
        pltpu.sync_copy(dstp_hbm.at[pl.ds(off, CE2)], dbuf)
        da = [pltpu.async_copy(ea2_hbm.at[pbuf.at[pl.ds(j * 128, 128)]],
                               ebuf.at[pl.ds(j * 128, 128)], sema)
              for j in range(KCH)]
        for dd in da:
            dd.wait()
        db = [pltpu.async_copy(h_hbm.at[sbuf.at[pl.ds(j * 128, 128)]],
                               ebuf.at[pl.ds(j * 128, 128)], semb, add=True)
              for j in range(KCH)]
        for dd in db:
            dd.wait()
        lax.fori_loop(0, CE2 // 16, grp, 0)
        return carry

    lax.fori_loop(0, nch, chunk, 0)

    @pl.when(n1 - n0 == 320)
    def _():
        pltpu.sync_copy(local.at[pl.ds(0, 320)], out_hbm.at[pl.ds(n0, 320)])

    @pl.when(n1 - n0 == 312)
    def _():
        pltpu.sync_copy(local.at[pl.ds(0, 312)], out_hbm.at[pl.ds(n0, 312)])


def kernel(vocab_datas, x, edge_index, edge_attr, inter_tensor, smiles_id, batch,
           emb_table, node_W, node_b, node_gamma, node_beta, atom_W, atom_b,
           gnn_W1, gnn_b1, gnn_W2, gnn_b2, gnn_gamma, gnn_beta, pred_W, pred_b):
    xs = jnp.squeeze(x)

    tbl_in = jnp.concatenate([emb_table, vocab_datas], axis=1)
    fused = _fused_table(tbl_in, node_W, node_b)
    xs_pad = jnp.pad(xs, (0, N_PAD - N))
    h0pre = _sc_gather(fused, xs_pad)[:N]
    h = _bn(jax.nn.relu(h0pre), node_gamma, node_beta)

    x36 = jnp.concatenate([inter_tensor.reshape(E, 2 * DI), edge_attr], axis=1)
    w_cat = jnp.zeros((36, D), jnp.float32)
    w_cat = w_cat.at[0:DI, 0:64].set(atom_W[0:DI])
    w_cat = w_cat.at[32:34, 0:64].set(atom_W[DI:DI + 2])
    w_cat = w_cat.at[DI:2 * DI, 64:128].set(atom_W[0:DI])
    w_cat = w_cat.at[34:36, 64:128].set(atom_W[DI:DI + 2])
    b_cat = jnp.concatenate([atom_b, atom_b], axis=0)
    ea2 = _edge_enc(x36, w_cat, b_cat).reshape(2 * E, D)

    src = edge_index[0]
    dst = edge_index[1]
    srcf = jnp.concatenate([src, dst]).astype(jnp.int32)
    dstf = jnp.concatenate([dst, src]).astype(jnp.int32)
    perm = jnp.argsort(dstf, stable=True).astype(jnp.int32)
    dstp = jnp.take(dstf, perm)
    srcp = jnp.take(srcf, perm)
    perm_p = jnp.concatenate([perm, jnp.zeros((EPAD,), jnp.int32)])
    srcp_p = jnp.concatenate([srcp, jnp.zeros((EPAD,), jnp.int32)])
    dstp_p = jnp.concatenate([dstp, jnp.full((EPAD,), N, jnp.int32)])
    nbs = jnp.asarray([8 * ((w * (N // 8)) // NW) for w in range(NW + 1)],
                      jnp.int32)
    est = (jnp.searchsorted(dstp, nbs[:NW]).astype(jnp.int32) // 8) * 8
    eend = jnp.searchsorted(dstp, nbs[1:]).astype(jnp.int32)
    ebound = jnp.zeros((NW * 8,), jnp.int32)
    ebound = ebound.at[0::8].set(est).at[1::8].set(eend)

    for i in range(L):
        agg = _sc_mp(h, ea2, perm_p, srcp_p, dstp_p, ebound)
        u = _mlp(h, agg, gnn_W1[i], gnn_b1[i], gnn_W2[i], gnn_b2[i])
        u = _bn(u, gnn_gamma[i], gnn_beta[i])
        if i < L - 1:
            u = jax.nn.relu(u)
        h = u

    node_representation, mol_pred = _final(h, pred_W, pred_b, batch)
    return (mol_pred, node_representation)
